# baseline (speedup 1.0000x reference)
import functools

import jax
import jax.numpy as jnp
from jax.experimental import pallas as pl
from jax.experimental.pallas import tpu as pltpu


def _proj_body(x_ref, w_ref, o_ref):
    o_ref[...] = jnp.dot(x_ref[...], w_ref[...],
                         preferred_element_type=jnp.float32)


def _spmm_body(s_ref, p_ref, b_ref, o_ref, *, bk: int):
    k = pl.program_id(1)
    p_blk = p_ref[pl.ds(k * bk, bk), :]
    acc = jnp.dot(s_ref[...], p_blk, preferred_element_type=jnp.float32)

    @pl.when(k == 0)
    def _init():
        o_ref[...] = acc + b_ref[...]

    @pl.when(k != 0)
    def _accum():
        o_ref[...] += acc


def kernel(x, support, W0, b):
    n, d_in = x.shape
    d_out = W0.shape[1]
    a = support[0]

    bm = min(512, n)
    bk = min(1024, n)
    bx = min(1024, n)

    pre_sup = pl.pallas_call(
        _proj_body,
        grid=(n // bx,),
        in_specs=[
            pl.BlockSpec((bx, d_in), lambda i: (i, 0)),
            pl.BlockSpec((d_in, d_out), lambda i: (0, 0)),
        ],
        out_specs=pl.BlockSpec((bx, d_out), lambda i: (i, 0)),
        out_shape=jax.ShapeDtypeStruct((n, d_out), jnp.float32),
    )(x, W0)

    out = pl.pallas_call(
        functools.partial(_spmm_body, bk=bk),
        grid=(n // bm, n // bk),
        in_specs=[
            pl.BlockSpec((bm, bk), lambda i, k: (i, k)),
            pl.BlockSpec((n, d_out), lambda i, k: (0, 0)),
            pl.BlockSpec((1, d_out), lambda i, k: (0, 0)),
        ],
        out_specs=pl.BlockSpec((bm, d_out), lambda i, k: (i, 0)),
        out_shape=jax.ShapeDtypeStruct((n, d_out), jnp.float32),
        compiler_params=pltpu.CompilerParams(
            dimension_semantics=("parallel", "arbitrary"),
        ),
    )(a, pre_sup, b)
    return out
